# baseline (speedup 1.0000x reference)
import functools

import jax
import jax.numpy as jnp
from jax import lax
from jax.experimental import pallas as pl
from jax.experimental.pallas import tpu as pltpu
from jax.experimental.pallas import tpu_sc as plsc

N = 10000
E = 320000
DIN = 128
HID = 32
H1 = 8
DOUT = 128
NG = 64

NC = 2
NS = 16
NW = NC * NS
NPT = 320
NPAD = NW * NPT
CE = 4000
B = 16


def _leaky(v):
    return jnp.maximum(v, 0.2 * v)


def _tc1_body(x_ref, wl_ref, bl_ref, wr_ref, br_ref, a1_ref, xl_out, xr_out):
    xb = x_ref[...]
    xl = jnp.dot(xb, wl_ref[...], preferred_element_type=jnp.float32) + bl_ref[...]
    xr = jnp.dot(xb, wr_ref[...], preferred_element_type=jnp.float32) + br_ref[...]
    m = _leaky(xl + xr)
    s = jnp.dot(m, a1_ref[...], preferred_element_type=jnp.float32)
    r = xb.shape[0]
    xl_out[...] = xl
    xr_out[...] = jnp.concatenate([xr, s, jnp.zeros((r, 8), jnp.float32)], axis=1)


def _tc1(x, Wl1, bl1, Wr1, br1, A1):
    R = 1000
    grid = (N // R,)
    return pl.pallas_call(
        _tc1_body,
        grid=grid,
        in_specs=[
            pl.BlockSpec((R, DIN), lambda i: (i, 0)),
            pl.BlockSpec((DIN, H1 * HID), lambda i: (0, 0)),
            pl.BlockSpec((1, H1 * HID), lambda i: (0, 0)),
            pl.BlockSpec((DIN, H1 * HID), lambda i: (0, 0)),
            pl.BlockSpec((1, H1 * HID), lambda i: (0, 0)),
            pl.BlockSpec((H1 * HID, H1), lambda i: (0, 0)),
        ],
        out_specs=[
            pl.BlockSpec((R, H1 * HID), lambda i: (i, 0)),
            pl.BlockSpec((R, H1 * HID + 16), lambda i: (i, 0)),
        ],
        out_shape=[
            jax.ShapeDtypeStruct((N, H1 * HID), jnp.float32),
            jax.ShapeDtypeStruct((N, H1 * HID + 16), jnp.float32),
        ],
    )(x, Wl1, bl1, Wr1, br1, A1)


def _tc2_body(xl_ref, acc_ref, b1_ref, wl_ref, bl_ref, wr_ref, br_ref, a2_ref,
              xl2_out, xr2_out):
    C = H1 * HID
    xl = xl_ref[...]
    numer = acc_ref[:, :C]
    denomv = acc_ref[:, C:C + H1]
    r = xl.shape[0]
    denom_full = jnp.concatenate(
        [jnp.broadcast_to(denomv[:, h:h + 1], (r, HID)) for h in range(H1)],
        axis=1)
    h1 = (xl + numer) / (1.0 + denom_full) + b1_ref[...]
    h1 = jnp.where(h1 > 0, h1, jnp.exp(jnp.minimum(h1, 0.0)) - 1.0)
    xl2 = jnp.dot(h1, wl_ref[...], preferred_element_type=jnp.float32) + bl_ref[...]
    xr2 = jnp.dot(h1, wr_ref[...], preferred_element_type=jnp.float32) + br_ref[...]
    m2 = _leaky(xl2 + xr2)
    s2 = jnp.dot(m2, a2_ref[...], preferred_element_type=jnp.float32)
    xl2_out[...] = xl2
    xr2_out[...] = jnp.concatenate([xr2, s2, jnp.zeros((r, 15), jnp.float32)], axis=1)


def _tc2(xl1, acc1, bias1, Wl2, bl2, Wr2, br2, A2):
    R = 1000
    C = H1 * HID
    grid = (N // R,)
    return pl.pallas_call(
        _tc2_body,
        grid=grid,
        in_specs=[
            pl.BlockSpec((R, C), lambda i: (i, 0)),
            pl.BlockSpec((R, C + 16), lambda i: (i, 0)),
            pl.BlockSpec((1, C), lambda i: (0, 0)),
            pl.BlockSpec((C, DOUT), lambda i: (0, 0)),
            pl.BlockSpec((1, DOUT), lambda i: (0, 0)),
            pl.BlockSpec((C, DOUT), lambda i: (0, 0)),
            pl.BlockSpec((1, DOUT), lambda i: (0, 0)),
            pl.BlockSpec((DOUT, 1), lambda i: (0, 0)),
        ],
        out_specs=[
            pl.BlockSpec((R, DOUT), lambda i: (i, 0)),
            pl.BlockSpec((R, DOUT + 16), lambda i: (i, 0)),
        ],
        out_shape=[
            jax.ShapeDtypeStruct((N, DOUT), jnp.float32),
            jax.ShapeDtypeStruct((N, DOUT + 16), jnp.float32),
        ],
    )(xl1, acc1, bias1, Wl2, bl2, Wr2, br2, A2)


def _tc3_body(xl2_ref, acc_ref, b2_ref, batch_ref, out_ref, sums, cnt):
    step = pl.program_id(0)
    last = pl.num_programs(0) - 1

    @pl.when(step == 0)
    def _():
        sums[...] = jnp.zeros_like(sums)
        cnt[...] = jnp.zeros_like(cnt)

    xl2 = xl2_ref[...]
    r = xl2.shape[0]
    numer = acc_ref[:, :DOUT]
    denom = jnp.broadcast_to(acc_ref[:, DOUT:DOUT + 1], (r, DOUT))
    h2 = (xl2 + numer) / (1.0 + denom) + b2_ref[...]
    bv = batch_ref[0, 0, :]
    P = (bv[None, :] == lax.broadcasted_iota(jnp.int32, (NG, r), 0)
         ).astype(jnp.float32)
    sums[...] += jnp.dot(P, h2, preferred_element_type=jnp.float32)
    cnt[...] += jnp.broadcast_to(
        jnp.sum(P, axis=1, keepdims=True), (NG, DOUT))

    @pl.when(step == last)
    def _():
        out_ref[...] = sums[...] / jnp.maximum(cnt[...], 1.0)


def _tc3(xl2, acc2, bias2, batch3d):
    R = 1000
    grid = (N // R,)
    return pl.pallas_call(
        _tc3_body,
        grid=grid,
        in_specs=[
            pl.BlockSpec((R, DOUT), lambda i: (i, 0)),
            pl.BlockSpec((R, DOUT + 16), lambda i: (i, 0)),
            pl.BlockSpec((1, DOUT), lambda i: (0, 0)),
            pl.BlockSpec((1, 1, R), lambda i: (i, 0, 0)),
        ],
        out_specs=pl.BlockSpec((NG, DOUT), lambda i: (0, 0)),
        out_shape=jax.ShapeDtypeStruct((NG, DOUT), jnp.float32),
        scratch_shapes=[
            pltpu.VMEM((NG, DOUT), jnp.float32),
            pltpu.VMEM((NG, DOUT), jnp.float32),
        ],
    )(xl2, acc2, bias2, batch3d)


_GDN = lax.GatherDimensionNumbers(
    offset_dims=(), collapsed_slice_dims=(0,), start_index_map=(0,))


def _take16(v, j):
    idx = jnp.full((16, 1), j, jnp.int32)
    return lax.gather(v, idx, _GDN, (1,),
                      mode=lax.GatherScatterMode.PROMISE_IN_BOUNDS)


def _make_edge_kernel(C, CA, heads):
    hid = C // heads
    AW = C + 16

    mesh = plsc.VectorSubcoreMesh(core_axis_name="c", subcore_axis_name="s")

    @functools.partial(
        pl.kernel,
        out_type=jax.ShapeDtypeStruct((NPAD * AW,), jnp.float32),
        mesh=mesh,
        compiler_params=pltpu.CompilerParams(use_tc_tiling_on_sc=False,
                                             needs_layout_passes=False),
        scratch_types=[
            pltpu.VMEM((CE,), jnp.int32),
            pltpu.VMEM((CE,), jnp.int32),
            pltpu.VMEM((CE,), jnp.int32),
            pltpu.VMEM((CE,), jnp.int32),
            pltpu.VMEM((B, C), jnp.float32),
            pltpu.VMEM((B, CA), jnp.float32),
            pltpu.VMEM((B, C), jnp.float32),
            pltpu.VMEM((B, CA), jnp.float32),
            pltpu.VMEM((C,), jnp.float32),
            pltpu.VMEM((NPT * AW,), jnp.float32),
            pltpu.SemaphoreType.DMA,
            pltpu.SemaphoreType.DMA,
            pltpu.SemaphoreType.DMA,
            pltpu.SemaphoreType.DMA,
        ],
    )
    def edge_kernel(src_hbm, dst_hbm, xl_hbm, xr_hbm, att_hbm, out_hbm,
                    srcc, dstc, csrc, cdst, xlba, xrba, xlbb, xrbb, attb, acc,
                    sem1, sem2, sem3, sem4):
        wid = lax.axis_index("s") * NC + lax.axis_index("c")
        lo = wid * NPT
        lane = lax.iota(jnp.int32, 16)
        lov = jnp.broadcast_to(lo, (16,))
        hiv = jnp.broadcast_to(lo + NPT, (16,))
        zf = jnp.zeros((16,), jnp.float32)
        zi = jnp.zeros((16,), jnp.int32)
        nptm1 = jnp.full((16,), NPT - 1, jnp.int32)

        pltpu.sync_copy(att_hbm, attb)

        def zero_acc(r, _):
            acc[pl.ds(16 * r, 16)] = zf
            return 0

        lax.fori_loop(0, NPT * AW // 16, zero_acc, 0)

        def zero_idx(i, _):
            csrc[pl.ds(16 * i, 16)] = zi
            cdst[pl.ds(16 * i, 16)] = zi
            return 0

        lax.fori_loop(0, CE // 16, zero_idx, 0)

        def chunk_body(kc, _):
            cp1 = pltpu.async_copy(src_hbm.at[pl.ds(kc * CE, CE)], srcc, sem1)
            cp2 = pltpu.async_copy(dst_hbm.at[pl.ds(kc * CE, CE)], dstc, sem2)
            cp1.wait()
            cp2.wait()

            def filt(i, ptr):
                d = dstc[pl.ds(16 * i, 16)]
                msk = (d >= lov) & (d < hiv)
                cnt = jnp.sum(msk.astype(jnp.int32))
                plsc.store_compressed(csrc.at[pl.ds(ptr, 16)],
                                      srcc[pl.ds(16 * i, 16)], mask=msk)
                plsc.store_compressed(cdst.at[pl.ds(ptr, 16)], d, mask=msk)
                return ptr + cnt

            K = lax.fori_loop(0, CE // 16, filt, 0)
            Kv = jnp.broadcast_to(K, (16,))

            def issue(b0, xlbuf, xrbuf, s1, s2):
                pltpu.async_copy(xl_hbm.at[csrc.at[pl.ds(b0, B)]], xlbuf, s1)
                pltpu.async_copy(xr_hbm.at[cdst.at[pl.ds(b0, B)]], xrbuf, s2)

            def drain(xlbuf, xrbuf, s1, s2):
                pltpu.make_async_copy(xl_hbm.at[csrc.at[pl.ds(0, B)]],
                                      xlbuf, s1).wait()
                pltpu.make_async_copy(xr_hbm.at[cdst.at[pl.ds(0, B)]],
                                      xrbuf, s2).wait()

            def compute(b0, xlb, xrb):
                dvec = cdst[pl.ds(b0, B)]
                b0v = jnp.broadcast_to(b0, (16,))
                for b in range(B):
                    validv = (b0v + jnp.full((16,), b, jnp.int32)) < Kv
                    dlv = jnp.minimum(jnp.maximum(_take16(dvec, b) - lov, zi),
                                      nptm1)
                    rowbase = dlv * AW
                    lvec = jnp.zeros((16,), jnp.float32)
                    for h in range(heads):
                        t = jnp.zeros((16,), jnp.float32)
                        for q in range(hid // 16):
                            c0 = h * hid + 16 * q
                            xlv = xlb[b, pl.ds(c0, 16)]
                            xrv = xrb[b, pl.ds(c0, 16)]
                            av = attb[pl.ds(c0, 16)]
                            t = t + _leaky(xlv + xrv) * av
                        lgv = jnp.broadcast_to(jnp.sum(t), (16,))
                        lvec = jnp.where(lane == h, lgv, lvec)
                    svec = xrb[b, pl.ds(C, 16)]
                    w = jnp.where(validv, jnp.exp(lvec - svec), zf)
                    plsc.addupdate_scatter(acc, [rowbase + C + lane], w)
                    for h in range(heads):
                        whv = _take16(w, h)
                        for q in range(hid // 16):
                            c0 = h * hid + 16 * q
                            xlv = xlb[b, pl.ds(c0, 16)]
                            plsc.addupdate_scatter(
                                acc, [rowbase + c0 + lane], xlv * whv)

            nb2 = (K + 2 * B - 1) // (2 * B)
            issue(0, xlba, xrba, sem1, sem2)

            def pair_body(i, _):
                b0a = (2 * i) * B
                b0b = b0a + B
                b0c = jnp.minimum(b0a + 2 * B, CE - B)
                issue(b0b, xlbb, xrbb, sem3, sem4)
                drain(xlba, xrba, sem1, sem2)
                compute(b0a, xlba, xrba)
                issue(b0c, xlba, xrba, sem1, sem2)
                drain(xlbb, xrbb, sem3, sem4)
                compute(b0b, xlbb, xrbb)
                return 0

            lax.fori_loop(0, nb2, pair_body, 0)
            drain(xlba, xrba, sem1, sem2)
            return 0

        lax.fori_loop(0, E // CE, chunk_body, 0)

        pltpu.sync_copy(acc, out_hbm.at[pl.ds(lo * AW, NPT * AW)])

    return edge_kernel


_edge_kernel_l1 = _make_edge_kernel(H1 * HID, H1 * HID + 16, H1)
_edge_kernel_l2 = _make_edge_kernel(DOUT, DOUT + 16, 1)


def kernel(x, edge_index, batch, Wl1, bl1, Wr1, br1, att1, bias1,
           Wl2, bl2, Wr2, br2, att2, bias2):
    src = edge_index[0]
    dst = edge_index[1]

    A1 = (jnp.eye(H1, dtype=jnp.float32)[:, None, :]
          * att1[:, :, None]).reshape(H1 * HID, H1)
    A2 = att2.reshape(DOUT, 1)

    AW1 = H1 * HID + 16
    AW2 = DOUT + 16
    xl1, xr1aug = _tc1(x, Wl1, bl1.reshape(1, -1), Wr1, br1.reshape(1, -1), A1)
    acc1 = _edge_kernel_l1(src, dst, xl1, xr1aug, att1.reshape(-1))
    acc1 = acc1.reshape(NPAD, AW1)
    xl2, xr2aug = _tc2(xl1, acc1[:N], bias1.reshape(1, -1),
                       Wl2, bl2.reshape(1, -1), Wr2, br2.reshape(1, -1), A2)
    acc2 = _edge_kernel_l2(src, dst, xl2, xr2aug, att2.reshape(-1))
    acc2 = acc2.reshape(NPAD, AW2)
    out = _tc3(xl2, acc2[:N], bias2.reshape(1, -1), batch.reshape(N // 1000, 1, 1000))
    return out


# speedup vs baseline: 1.2525x; 1.2525x over previous
import functools

import jax
import jax.numpy as jnp
from jax import lax
from jax.experimental import pallas as pl
from jax.experimental.pallas import tpu as pltpu
from jax.experimental.pallas import tpu_sc as plsc

N = 10000
E = 320000
DIN = 128
HID = 32
H1 = 8
DOUT = 128
NG = 64

NC = 2
NS = 16
NW = NC * NS
NPT = 320
NPAD = NW * NPT
CE = 4000
B = 16


def _leaky(v):
    return jnp.maximum(v, 0.2 * v)


def _tc1_body(x_ref, wl_ref, bl_ref, wr_ref, br_ref, a1_ref, xl_out, xr_out):
    xb = x_ref[...]
    xl = jnp.dot(xb, wl_ref[...], preferred_element_type=jnp.float32) + bl_ref[...]
    xr = jnp.dot(xb, wr_ref[...], preferred_element_type=jnp.float32) + br_ref[...]
    m = _leaky(xl + xr)
    s = jnp.dot(m, a1_ref[...], preferred_element_type=jnp.float32)
    r = xb.shape[0]
    xl_out[...] = xl
    xr_out[...] = jnp.concatenate([xr, s, jnp.zeros((r, 8), jnp.float32)], axis=1)


def _tc1(x, Wl1, bl1, Wr1, br1, A1):
    R = 1000
    grid = (N // R,)
    return pl.pallas_call(
        _tc1_body,
        grid=grid,
        in_specs=[
            pl.BlockSpec((R, DIN), lambda i: (i, 0)),
            pl.BlockSpec((DIN, H1 * HID), lambda i: (0, 0)),
            pl.BlockSpec((1, H1 * HID), lambda i: (0, 0)),
            pl.BlockSpec((DIN, H1 * HID), lambda i: (0, 0)),
            pl.BlockSpec((1, H1 * HID), lambda i: (0, 0)),
            pl.BlockSpec((H1 * HID, H1), lambda i: (0, 0)),
        ],
        out_specs=[
            pl.BlockSpec((R, H1 * HID), lambda i: (i, 0)),
            pl.BlockSpec((R, H1 * HID + 16), lambda i: (i, 0)),
        ],
        out_shape=[
            jax.ShapeDtypeStruct((N, H1 * HID), jnp.float32),
            jax.ShapeDtypeStruct((N, H1 * HID + 16), jnp.float32),
        ],
    )(x, Wl1, bl1, Wr1, br1, A1)


def _tc2_body(xl_ref, acc_ref, b1_ref, wl_ref, bl_ref, wr_ref, br_ref, a2_ref,
              xl2_out, xr2_out):
    C = H1 * HID
    xl = xl_ref[...]
    numer = acc_ref[:, :C]
    denomv = acc_ref[:, C:C + H1]
    r = xl.shape[0]
    denom_full = jnp.concatenate(
        [jnp.broadcast_to(denomv[:, h:h + 1], (r, HID)) for h in range(H1)],
        axis=1)
    h1 = (xl + numer) / (1.0 + denom_full) + b1_ref[...]
    h1 = jnp.where(h1 > 0, h1, jnp.exp(jnp.minimum(h1, 0.0)) - 1.0)
    xl2 = jnp.dot(h1, wl_ref[...], preferred_element_type=jnp.float32) + bl_ref[...]
    xr2 = jnp.dot(h1, wr_ref[...], preferred_element_type=jnp.float32) + br_ref[...]
    m2 = _leaky(xl2 + xr2)
    s2 = jnp.dot(m2, a2_ref[...], preferred_element_type=jnp.float32)
    xl2_out[...] = xl2
    xr2_out[...] = jnp.concatenate([xr2, s2, jnp.zeros((r, 15), jnp.float32)], axis=1)


def _tc2(xl1, acc1, bias1, Wl2, bl2, Wr2, br2, A2):
    R = 1000
    C = H1 * HID
    grid = (N // R,)
    return pl.pallas_call(
        _tc2_body,
        grid=grid,
        in_specs=[
            pl.BlockSpec((R, C), lambda i: (i, 0)),
            pl.BlockSpec((R, C + 16), lambda i: (i, 0)),
            pl.BlockSpec((1, C), lambda i: (0, 0)),
            pl.BlockSpec((C, DOUT), lambda i: (0, 0)),
            pl.BlockSpec((1, DOUT), lambda i: (0, 0)),
            pl.BlockSpec((C, DOUT), lambda i: (0, 0)),
            pl.BlockSpec((1, DOUT), lambda i: (0, 0)),
            pl.BlockSpec((DOUT, 1), lambda i: (0, 0)),
        ],
        out_specs=[
            pl.BlockSpec((R, DOUT), lambda i: (i, 0)),
            pl.BlockSpec((R, DOUT + 16), lambda i: (i, 0)),
        ],
        out_shape=[
            jax.ShapeDtypeStruct((N, DOUT), jnp.float32),
            jax.ShapeDtypeStruct((N, DOUT + 16), jnp.float32),
        ],
    )(xl1, acc1, bias1, Wl2, bl2, Wr2, br2, A2)


def _tc3_body(xl2_ref, acc_ref, b2_ref, batch_ref, out_ref, sums, cnt):
    step = pl.program_id(0)
    last = pl.num_programs(0) - 1

    @pl.when(step == 0)
    def _():
        sums[...] = jnp.zeros_like(sums)
        cnt[...] = jnp.zeros_like(cnt)

    xl2 = xl2_ref[...]
    r = xl2.shape[0]
    numer = acc_ref[:, :DOUT]
    denom = jnp.broadcast_to(acc_ref[:, DOUT:DOUT + 1], (r, DOUT))
    h2 = (xl2 + numer) / (1.0 + denom) + b2_ref[...]
    bv = batch_ref[0, 0, :]
    P = (bv[None, :] == lax.broadcasted_iota(jnp.int32, (NG, r), 0)
         ).astype(jnp.float32)
    sums[...] += jnp.dot(P, h2, preferred_element_type=jnp.float32)
    cnt[...] += jnp.broadcast_to(
        jnp.sum(P, axis=1, keepdims=True), (NG, DOUT))

    @pl.when(step == last)
    def _():
        out_ref[...] = sums[...] / jnp.maximum(cnt[...], 1.0)


def _tc3(xl2, acc2, bias2, batch3d):
    R = 1000
    grid = (N // R,)
    return pl.pallas_call(
        _tc3_body,
        grid=grid,
        in_specs=[
            pl.BlockSpec((R, DOUT), lambda i: (i, 0)),
            pl.BlockSpec((R, DOUT + 16), lambda i: (i, 0)),
            pl.BlockSpec((1, DOUT), lambda i: (0, 0)),
            pl.BlockSpec((1, 1, R), lambda i: (i, 0, 0)),
        ],
        out_specs=pl.BlockSpec((NG, DOUT), lambda i: (0, 0)),
        out_shape=jax.ShapeDtypeStruct((NG, DOUT), jnp.float32),
        scratch_shapes=[
            pltpu.VMEM((NG, DOUT), jnp.float32),
            pltpu.VMEM((NG, DOUT), jnp.float32),
        ],
    )(xl2, acc2, bias2, batch3d)


_GDN = lax.GatherDimensionNumbers(
    offset_dims=(), collapsed_slice_dims=(0,), start_index_map=(0,))


def _take16(v, j):
    idx = jnp.full((16, 1), j, jnp.int32)
    return lax.gather(v, idx, _GDN, (1,),
                      mode=lax.GatherScatterMode.PROMISE_IN_BOUNDS)


def _make_edge_kernel(C, CA, heads):
    hid = C // heads
    AW = C + 16

    mesh = plsc.VectorSubcoreMesh(core_axis_name="c", subcore_axis_name="s")

    @functools.partial(
        pl.kernel,
        out_type=jax.ShapeDtypeStruct((NPAD * AW,), jnp.float32),
        mesh=mesh,
        compiler_params=pltpu.CompilerParams(use_tc_tiling_on_sc=False,
                                             needs_layout_passes=False),
        scratch_types=[
            pltpu.VMEM((CE,), jnp.int32),
            pltpu.VMEM((CE,), jnp.int32),
            pltpu.VMEM((CE,), jnp.int32),
            pltpu.VMEM((CE,), jnp.int32),
            pltpu.VMEM((B, C), jnp.float32),
            pltpu.VMEM((B, CA), jnp.float32),
            pltpu.VMEM((C,), jnp.float32),
            pltpu.VMEM((NPT * AW,), jnp.float32),
            pltpu.SemaphoreType.DMA,
            pltpu.SemaphoreType.DMA,
        ],
    )
    def edge_kernel(src_hbm, dst_hbm, xl_hbm, xr_hbm, att_hbm, out_hbm,
                    srcc, dstc, csrc, cdst, xlb, xrb, attb, acc,
                    sem1, sem2):
        wid = lax.axis_index("s") * NC + lax.axis_index("c")
        lo = wid * NPT
        lane = lax.iota(jnp.int32, 16)
        lov = jnp.broadcast_to(lo, (16,))
        hiv = jnp.broadcast_to(lo + NPT, (16,))
        zf = jnp.zeros((16,), jnp.float32)
        zi = jnp.zeros((16,), jnp.int32)
        nptm1 = jnp.full((16,), NPT - 1, jnp.int32)

        pltpu.sync_copy(att_hbm, attb)

        avs = [attb[pl.ds(16 * q, 16)] for q in range(C // 16)]
        hmasks = [lane == jnp.full((16,), h, jnp.int32) for h in range(heads)]
        laneoff = [lane + jnp.full((16,), c0, jnp.int32)
                   for c0 in range(0, C + 16, 16)]

        def zero_acc(r, _):
            acc[pl.ds(16 * r, 16)] = zf
            return 0

        lax.fori_loop(0, NPT * AW // 16, zero_acc, 0)

        def zero_idx(i, _):
            csrc[pl.ds(16 * i, 16)] = zi
            cdst[pl.ds(16 * i, 16)] = zi
            return 0

        lax.fori_loop(0, CE // 16, zero_idx, 0)

        def chunk_body(kc, _):
            cp1 = pltpu.async_copy(src_hbm.at[pl.ds(kc * CE, CE)], srcc, sem1)
            cp2 = pltpu.async_copy(dst_hbm.at[pl.ds(kc * CE, CE)], dstc, sem2)
            cp1.wait()
            cp2.wait()

            def filt(i, ptr):
                d = dstc[pl.ds(16 * i, 16)]
                msk = (d >= lov) & (d < hiv)
                cnt = jnp.sum(msk.astype(jnp.int32))
                plsc.store_compressed(csrc.at[pl.ds(ptr, 16)],
                                      srcc[pl.ds(16 * i, 16)], mask=msk)
                plsc.store_compressed(cdst.at[pl.ds(ptr, 16)], d, mask=msk)
                return ptr + cnt

            K = lax.fori_loop(0, CE // 16, filt, 0)
            Kv = jnp.broadcast_to(K, (16,))

            nb = (K + B - 1) // B

            def batch_body(j, _):
                b0 = j * B
                g1 = pltpu.async_copy(xl_hbm.at[csrc.at[pl.ds(b0, B)]],
                                      xlb, sem1)
                g2 = pltpu.async_copy(xr_hbm.at[cdst.at[pl.ds(b0, B)]],
                                      xrb, sem2)
                g1.wait()
                g2.wait()

                dvec = cdst[pl.ds(b0, B)]
                b0v = jnp.broadcast_to(b0, (16,))
                for b in range(B):
                    validv = (b0v + jnp.full((16,), b, jnp.int32)) < Kv
                    dlv = jnp.minimum(jnp.maximum(_take16(dvec, b) - lov, zi),
                                      nptm1)
                    rowbase = dlv * AW
                    lvec = jnp.zeros((16,), jnp.float32)
                    for h in range(heads):
                        t = jnp.zeros((16,), jnp.float32)
                        for q in range(hid // 16):
                            c0 = h * hid + 16 * q
                            t = t + _leaky(xlb[b, pl.ds(c0, 16)]
                                           + xrb[b, pl.ds(c0, 16)]) \
                                * avs[(h * hid + 16 * q) // 16]
                        lgv = jnp.broadcast_to(jnp.sum(t), (16,))
                        lvec = jnp.where(hmasks[h], lgv, lvec)
                    svec = xrb[b, pl.ds(C, 16)]
                    w = jnp.where(validv, jnp.exp(lvec - svec), zf)
                    plsc.addupdate_scatter(acc, [rowbase + laneoff[C // 16]], w)
                    for h in range(heads):
                        whv = _take16(w, h)
                        for q in range(hid // 16):
                            c0 = h * hid + 16 * q
                            plsc.addupdate_scatter(
                                acc, [rowbase + laneoff[c0 // 16]],
                                xlb[b, pl.ds(c0, 16)] * whv)
                return 0

            lax.fori_loop(0, nb, batch_body, 0)
            return 0

        lax.fori_loop(0, E // CE, chunk_body, 0)

        pltpu.sync_copy(acc, out_hbm.at[pl.ds(lo * AW, NPT * AW)])

    return edge_kernel


_edge_kernel_l1 = _make_edge_kernel(H1 * HID, H1 * HID + 16, H1)
_edge_kernel_l2 = _make_edge_kernel(DOUT, DOUT + 16, 1)


def kernel(x, edge_index, batch, Wl1, bl1, Wr1, br1, att1, bias1,
           Wl2, bl2, Wr2, br2, att2, bias2):
    src = edge_index[0]
    dst = edge_index[1]

    A1 = (jnp.eye(H1, dtype=jnp.float32)[:, None, :]
          * att1[:, :, None]).reshape(H1 * HID, H1)
    A2 = att2.reshape(DOUT, 1)

    AW1 = H1 * HID + 16
    AW2 = DOUT + 16
    xl1, xr1aug = _tc1(x, Wl1, bl1.reshape(1, -1), Wr1, br1.reshape(1, -1), A1)
    acc1 = _edge_kernel_l1(src, dst, xl1, xr1aug, att1.reshape(-1))
    acc1 = acc1.reshape(NPAD, AW1)
    xl2, xr2aug = _tc2(xl1, acc1[:N], bias1.reshape(1, -1),
                       Wl2, bl2.reshape(1, -1), Wr2, br2.reshape(1, -1), A2)
    acc2 = _edge_kernel_l2(src, dst, xl2, xr2aug, att2.reshape(-1))
    acc2 = acc2.reshape(NPAD, AW2)
    out = _tc3(xl2, acc2[:N], bias2.reshape(1, -1), batch.reshape(N // 1000, 1, 1000))
    return out


# speedup vs baseline: 1.2576x; 1.0041x over previous
import functools

import jax
import jax.numpy as jnp
from jax import lax
from jax.experimental import pallas as pl
from jax.experimental.pallas import tpu as pltpu
from jax.experimental.pallas import tpu_sc as plsc

N = 10000
E = 320000
DIN = 128
HID = 32
H1 = 8
DOUT = 128
NG = 64

NC = 2
NS = 16
NW = NC * NS
NPT = 320
NPAD = NW * NPT
CE = 4000
B = 16


def _leaky(v):
    return jnp.maximum(v, 0.2 * v)


def _tc1_body(x_ref, wl_ref, bl_ref, wr_ref, br_ref, a1_ref, xl_out, xr_out):
    xb = x_ref[...]
    xl = jnp.dot(xb, wl_ref[...], preferred_element_type=jnp.float32) + bl_ref[...]
    xr = jnp.dot(xb, wr_ref[...], preferred_element_type=jnp.float32) + br_ref[...]
    m = _leaky(xl + xr)
    s = jnp.dot(m, a1_ref[...], preferred_element_type=jnp.float32)
    r = xb.shape[0]
    xl_out[...] = xl
    xr_out[...] = jnp.concatenate([xr, s, jnp.zeros((r, 8), jnp.float32)], axis=1)


def _tc1(x, Wl1, bl1, Wr1, br1, A1):
    R = 1000
    grid = (N // R,)
    return pl.pallas_call(
        _tc1_body,
        grid=grid,
        in_specs=[
            pl.BlockSpec((R, DIN), lambda i: (i, 0)),
            pl.BlockSpec((DIN, H1 * HID), lambda i: (0, 0)),
            pl.BlockSpec((1, H1 * HID), lambda i: (0, 0)),
            pl.BlockSpec((DIN, H1 * HID), lambda i: (0, 0)),
            pl.BlockSpec((1, H1 * HID), lambda i: (0, 0)),
            pl.BlockSpec((H1 * HID, H1), lambda i: (0, 0)),
        ],
        out_specs=[
            pl.BlockSpec((R, H1 * HID), lambda i: (i, 0)),
            pl.BlockSpec((R, H1 * HID + 16), lambda i: (i, 0)),
        ],
        out_shape=[
            jax.ShapeDtypeStruct((N, H1 * HID), jnp.float32),
            jax.ShapeDtypeStruct((N, H1 * HID + 16), jnp.float32),
        ],
    )(x, Wl1, bl1, Wr1, br1, A1)


def _tc2_body(xl_ref, acc_ref, b1_ref, wl_ref, bl_ref, wr_ref, br_ref, a2_ref,
              xl2_out, xr2_out):
    C = H1 * HID
    xl = xl_ref[...]
    numer = acc_ref[:, :C]
    denomv = acc_ref[:, C:C + H1]
    r = xl.shape[0]
    denom_full = jnp.concatenate(
        [jnp.broadcast_to(denomv[:, h:h + 1], (r, HID)) for h in range(H1)],
        axis=1)
    h1 = (xl + numer) / (1.0 + denom_full) + b1_ref[...]
    h1 = jnp.where(h1 > 0, h1, jnp.exp(jnp.minimum(h1, 0.0)) - 1.0)
    xl2 = jnp.dot(h1, wl_ref[...], preferred_element_type=jnp.float32) + bl_ref[...]
    xr2 = jnp.dot(h1, wr_ref[...], preferred_element_type=jnp.float32) + br_ref[...]
    m2 = _leaky(xl2 + xr2)
    s2 = jnp.dot(m2, a2_ref[...], preferred_element_type=jnp.float32)
    xl2_out[...] = xl2
    xr2_out[...] = jnp.concatenate([xr2, s2, jnp.zeros((r, 15), jnp.float32)], axis=1)


def _tc2(xl1, acc1, bias1, Wl2, bl2, Wr2, br2, A2):
    R = 1000
    C = H1 * HID
    grid = (N // R,)
    return pl.pallas_call(
        _tc2_body,
        grid=grid,
        in_specs=[
            pl.BlockSpec((R, C), lambda i: (i, 0)),
            pl.BlockSpec((R, C + 16), lambda i: (i, 0)),
            pl.BlockSpec((1, C), lambda i: (0, 0)),
            pl.BlockSpec((C, DOUT), lambda i: (0, 0)),
            pl.BlockSpec((1, DOUT), lambda i: (0, 0)),
            pl.BlockSpec((C, DOUT), lambda i: (0, 0)),
            pl.BlockSpec((1, DOUT), lambda i: (0, 0)),
            pl.BlockSpec((DOUT, 1), lambda i: (0, 0)),
        ],
        out_specs=[
            pl.BlockSpec((R, DOUT), lambda i: (i, 0)),
            pl.BlockSpec((R, DOUT + 16), lambda i: (i, 0)),
        ],
        out_shape=[
            jax.ShapeDtypeStruct((N, DOUT), jnp.float32),
            jax.ShapeDtypeStruct((N, DOUT + 16), jnp.float32),
        ],
    )(xl1, acc1, bias1, Wl2, bl2, Wr2, br2, A2)


def _tc3_body(xl2_ref, acc_ref, b2_ref, batch_ref, out_ref, sums, cnt):
    step = pl.program_id(0)
    last = pl.num_programs(0) - 1

    @pl.when(step == 0)
    def _():
        sums[...] = jnp.zeros_like(sums)
        cnt[...] = jnp.zeros_like(cnt)

    xl2 = xl2_ref[...]
    r = xl2.shape[0]
    numer = acc_ref[:, :DOUT]
    denom = jnp.broadcast_to(acc_ref[:, DOUT:DOUT + 1], (r, DOUT))
    h2 = (xl2 + numer) / (1.0 + denom) + b2_ref[...]
    bv = batch_ref[0, 0, :]
    P = (bv[None, :] == lax.broadcasted_iota(jnp.int32, (NG, r), 0)
         ).astype(jnp.float32)
    sums[...] += jnp.dot(P, h2, preferred_element_type=jnp.float32)
    cnt[...] += jnp.broadcast_to(
        jnp.sum(P, axis=1, keepdims=True), (NG, DOUT))

    @pl.when(step == last)
    def _():
        out_ref[...] = sums[...] / jnp.maximum(cnt[...], 1.0)


def _tc3(xl2, acc2, bias2, batch3d):
    R = 1000
    grid = (N // R,)
    return pl.pallas_call(
        _tc3_body,
        grid=grid,
        in_specs=[
            pl.BlockSpec((R, DOUT), lambda i: (i, 0)),
            pl.BlockSpec((R, DOUT + 16), lambda i: (i, 0)),
            pl.BlockSpec((1, DOUT), lambda i: (0, 0)),
            pl.BlockSpec((1, 1, R), lambda i: (i, 0, 0)),
        ],
        out_specs=pl.BlockSpec((NG, DOUT), lambda i: (0, 0)),
        out_shape=jax.ShapeDtypeStruct((NG, DOUT), jnp.float32),
        scratch_shapes=[
            pltpu.VMEM((NG, DOUT), jnp.float32),
            pltpu.VMEM((NG, DOUT), jnp.float32),
        ],
    )(xl2, acc2, bias2, batch3d)


_GDN = lax.GatherDimensionNumbers(
    offset_dims=(), collapsed_slice_dims=(0,), start_index_map=(0,))


def _take16(v, j):
    idx = jnp.full((16, 1), j, jnp.int32)
    return lax.gather(v, idx, _GDN, (1,),
                      mode=lax.GatherScatterMode.PROMISE_IN_BOUNDS)


def _make_edge_kernel(C, CA, heads):
    hid = C // heads
    AW = C + 16

    mesh = plsc.VectorSubcoreMesh(core_axis_name="c", subcore_axis_name="s")

    @functools.partial(
        pl.kernel,
        out_type=jax.ShapeDtypeStruct((NPAD * AW,), jnp.float32),
        mesh=mesh,
        compiler_params=pltpu.CompilerParams(use_tc_tiling_on_sc=False,
                                             needs_layout_passes=False),
        scratch_types=[
            pltpu.VMEM((CE,), jnp.int32),
            pltpu.VMEM((CE,), jnp.int32),
            pltpu.VMEM((CE,), jnp.int32),
            pltpu.VMEM((CE,), jnp.int32),
            pltpu.VMEM((B, C), jnp.float32),
            pltpu.VMEM((B, CA), jnp.float32),
            pltpu.VMEM((C,), jnp.float32),
            pltpu.VMEM((NPT * AW,), jnp.float32),
            pltpu.SemaphoreType.DMA,
            pltpu.SemaphoreType.DMA,
            pltpu.SemaphoreType.DMA,
            pltpu.SemaphoreType.DMA,
        ],
    )
    def edge_kernel(src_hbm, dst_hbm, xl_hbm, xr_hbm, att_hbm, out_hbm,
                    srcc, dstc, csrc, cdst, xlb, xrb, attb, acc,
                    sem1, sem2, semA, semB):
        wid = lax.axis_index("s") * NC + lax.axis_index("c")
        lo = wid * NPT
        lane = lax.iota(jnp.int32, 16)
        lov = jnp.broadcast_to(lo, (16,))
        hiv = jnp.broadcast_to(lo + NPT, (16,))
        zf = jnp.zeros((16,), jnp.float32)
        zi = jnp.zeros((16,), jnp.int32)
        nptm1 = jnp.full((16,), NPT - 1, jnp.int32)

        pltpu.sync_copy(att_hbm, attb)

        avs = [attb[pl.ds(16 * q, 16)] for q in range(C // 16)]
        hmasks = [lane == jnp.full((16,), h, jnp.int32) for h in range(heads)]
        laneoff = [lane + jnp.full((16,), c0, jnp.int32)
                   for c0 in range(0, C + 16, 16)]

        def zero_acc(r, _):
            acc[pl.ds(16 * r, 16)] = zf
            return 0

        lax.fori_loop(0, NPT * AW // 16, zero_acc, 0)

        def zero_idx(i, _):
            csrc[pl.ds(16 * i, 16)] = zi
            cdst[pl.ds(16 * i, 16)] = zi
            return 0

        lax.fori_loop(0, CE // 16, zero_idx, 0)

        pltpu.async_copy(src_hbm.at[pl.ds(0, CE)], srcc, semA)
        pltpu.async_copy(dst_hbm.at[pl.ds(0, CE)], dstc, semB)

        def chunk_body(kc, _):
            pltpu.make_async_copy(src_hbm.at[pl.ds(0, CE)], srcc, semA).wait()
            pltpu.make_async_copy(dst_hbm.at[pl.ds(0, CE)], dstc, semB).wait()

            def filt(i, ptr):
                d = dstc[pl.ds(16 * i, 16)]
                msk = (d >= lov) & (d < hiv)
                cnt = plsc.all_reduce_population_count(msk)[0]
                plsc.store_compressed(csrc.at[pl.ds(ptr, 16)],
                                      srcc[pl.ds(16 * i, 16)], mask=msk)
                plsc.store_compressed(cdst.at[pl.ds(ptr, 16)], d, mask=msk)
                return ptr + cnt

            K = lax.fori_loop(0, CE // 16, filt, 0)

            kn = jnp.minimum(kc + 1, E // CE - 1)
            pltpu.async_copy(src_hbm.at[pl.ds(kn * CE, CE)], srcc, semA)
            pltpu.async_copy(dst_hbm.at[pl.ds(kn * CE, CE)], dstc, semB)
            Kv = jnp.broadcast_to(K, (16,))

            nb = (K + B - 1) // B

            def batch_body(j, _):
                b0 = j * B
                g1 = pltpu.async_copy(xl_hbm.at[csrc.at[pl.ds(b0, B)]],
                                      xlb, sem1)
                g2 = pltpu.async_copy(xr_hbm.at[cdst.at[pl.ds(b0, B)]],
                                      xrb, sem2)
                g1.wait()
                g2.wait()

                dvec = cdst[pl.ds(b0, B)]
                b0v = jnp.broadcast_to(b0, (16,))
                for b in range(B):
                    validv = (b0v + jnp.full((16,), b, jnp.int32)) < Kv
                    dlv = jnp.minimum(jnp.maximum(_take16(dvec, b) - lov, zi),
                                      nptm1)
                    rowbase = dlv * AW
                    lvec = jnp.zeros((16,), jnp.float32)
                    for h in range(heads):
                        t = jnp.zeros((16,), jnp.float32)
                        for q in range(hid // 16):
                            c0 = h * hid + 16 * q
                            t = t + _leaky(xlb[b, pl.ds(c0, 16)]
                                           + xrb[b, pl.ds(c0, 16)]) \
                                * avs[(h * hid + 16 * q) // 16]
                        lgv = jnp.broadcast_to(jnp.sum(t), (16,))
                        lvec = jnp.where(hmasks[h], lgv, lvec)
                    svec = xrb[b, pl.ds(C, 16)]
                    w = jnp.where(validv, jnp.exp(lvec - svec), zf)
                    plsc.addupdate_scatter(acc, [rowbase + laneoff[C // 16]], w)
                    for h in range(heads):
                        whv = _take16(w, h)
                        for q in range(hid // 16):
                            c0 = h * hid + 16 * q
                            plsc.addupdate_scatter(
                                acc, [rowbase + laneoff[c0 // 16]],
                                xlb[b, pl.ds(c0, 16)] * whv)
                return 0

            lax.fori_loop(0, nb, batch_body, 0)
            return 0

        lax.fori_loop(0, E // CE, chunk_body, 0)

        pltpu.make_async_copy(src_hbm.at[pl.ds(0, CE)], srcc, semA).wait()
        pltpu.make_async_copy(dst_hbm.at[pl.ds(0, CE)], dstc, semB).wait()

        pltpu.sync_copy(acc, out_hbm.at[pl.ds(lo * AW, NPT * AW)])

    return edge_kernel


_edge_kernel_l1 = _make_edge_kernel(H1 * HID, H1 * HID + 16, H1)
_edge_kernel_l2 = _make_edge_kernel(DOUT, DOUT + 16, 1)


def kernel(x, edge_index, batch, Wl1, bl1, Wr1, br1, att1, bias1,
           Wl2, bl2, Wr2, br2, att2, bias2):
    src = edge_index[0]
    dst = edge_index[1]

    A1 = (jnp.eye(H1, dtype=jnp.float32)[:, None, :]
          * att1[:, :, None]).reshape(H1 * HID, H1)
    A2 = att2.reshape(DOUT, 1)

    AW1 = H1 * HID + 16
    AW2 = DOUT + 16
    xl1, xr1aug = _tc1(x, Wl1, bl1.reshape(1, -1), Wr1, br1.reshape(1, -1), A1)
    acc1 = _edge_kernel_l1(src, dst, xl1, xr1aug, att1.reshape(-1))
    acc1 = acc1.reshape(NPAD, AW1)
    xl2, xr2aug = _tc2(xl1, acc1[:N], bias1.reshape(1, -1),
                       Wl2, bl2.reshape(1, -1), Wr2, br2.reshape(1, -1), A2)
    acc2 = _edge_kernel_l2(src, dst, xl2, xr2aug, att2.reshape(-1))
    acc2 = acc2.reshape(NPAD, AW2)
    out = _tc3(xl2, acc2[:N], bias2.reshape(1, -1), batch.reshape(N // 1000, 1, 1000))
    return out
